# baseline (speedup 1.0000x reference)
import math

import jax
import jax.numpy as jnp
from jax import lax
from jax.experimental import pallas as pl

_N_HEADS = 16
_D_MODEL = 1024
_N_CLUSTERS = 256
_BITS = 32


def _qkv_kernel(x_ref, wq_ref, wk_ref, wv_ref, b_ref, o_ref):
    x = x_ref[...]
    D = x.shape[1]
    o_ref[:, 0:D] = _dot_t(x, wq_ref[...]) + b_ref[:, 0:D]
    o_ref[:, D:2 * D] = _dot_t(x, wk_ref[...]) + b_ref[:, D:2 * D]
    o_ref[:, 2 * D:3 * D] = _dot_t(x, wv_ref[...]) + b_ref[:, 2 * D:3 * D]


def _dot(a, b):
    return lax.dot_general(a, b, (((1,), (0,)), ((), ())),
                           preferred_element_type=jnp.float32)


def _dot_t(a, b):
    return lax.dot_general(a, b, (((1,), (1,)), ((), ())),
                           preferred_element_type=jnp.float32)


def _cluster_attn_kernel(q_ref, k_ref, v_ref, pt_ref, ohinit_ref, o_ref):
    L = q_ref.shape[0]
    E = q_ref.shape[3]
    C = _N_CLUSTERS
    CH = 2048
    NCH = L // CH
    f32 = jnp.float32
    pt = pt_ref[...]

    bits_all = (_dot(q_ref[:, 0, 0, :], pt) > 0).astype(f32)
    cent = _dot(ohinit_ref[...], bits_all)

    iota_sc = lax.broadcasted_iota(jnp.int32, (CH, C), 1)
    iota_f = iota_sc.astype(f32)
    ones_col = jnp.ones((CH, 1), f32)

    def cs_row(centroids):
        return lax.dot_general(jnp.ones((1, _BITS), f32), centroids,
                               (((1,), (1,)), ((), ())),
                               preferred_element_type=f32)

    def onehot_chunk(i, centroids, csr):
        qc = q_ref[pl.ds(i * CH, CH), 0, 0, :]
        b = (_dot(qc, pt) > 0).astype(f32)
        xc = _dot_t(b, centroids)
        dd = (csr - 2.0 * xc) * 256.0 + iota_f
        mn = jnp.min(dd, axis=1, keepdims=True)
        return (dd == mn).astype(f32), b, qc

    csr1 = cs_row(cent)

    def body1(i, acc):
        oh, b, _ = onehot_chunk(i, cent, csr1)
        rhs = jnp.concatenate([b, ones_col], axis=1)
        return acc + lax.dot_general(oh, rhs, (((0,), (0,)), ((), ())),
                                     preferred_element_type=f32)

    acc1 = lax.fori_loop(0, NCH, body1, jnp.zeros((C, _BITS + 1), f32))
    sums1 = acc1[:, :_BITS]
    counts1 = acc1[:, _BITS:]
    cent2 = jnp.where(counts1 > 0, (2.0 * sums1 > counts1).astype(f32), cent)

    csr2 = cs_row(cent2)

    def body2(i, carry):
        acc, cnt_row = carry
        oh, _, qc = onehot_chunk(i, cent2, csr2)
        rhs = jnp.concatenate([qc, ones_col], axis=1)
        acc = acc + lax.dot_general(oh, rhs, (((0,), (0,)), ((), ())),
                                    preferred_element_type=f32)
        return acc, cnt_row + jnp.sum(oh, axis=0, keepdims=True)

    acc2, cnt_row = lax.fori_loop(
        0, NCH, body2,
        (jnp.zeros((C, E + 1), f32), jnp.zeros((1, C), f32)))
    qgsum = acc2[:, :E]
    counts2 = acc2[:, E:]
    factors = jnp.where(counts2 > 0, 1.0 / jnp.maximum(counts2, 1.0), 0.0)
    qg = qgsum * factors

    logits = _dot_t(qg, k_ref[:, 0, 0, :]) * (1.0 / math.sqrt(E))
    m = jnp.max(logits, axis=1, keepdims=True)
    e = jnp.exp(logits - m)
    attn = e / jnp.sum(e, axis=1, keepdims=True)
    vc = _dot(attn, v_ref[:, 0, 0, :])

    tri = (lax.broadcasted_iota(jnp.int32, (C, C), 0)
           <= lax.broadcasted_iota(jnp.int32, (C, C), 1)).astype(f32)
    cum = _dot(cnt_row, tri).astype(jnp.int32)

    def body3(i, carry):
        base = i * CH
        li = lax.broadcasted_iota(jnp.int32, (CH, C), 0) + base
        sc = jnp.sum((cum <= li).astype(jnp.int32), axis=1)
        oh3 = (sc[:, None] == iota_sc).astype(f32)
        o_ref[0, pl.ds(base, CH), :] = _dot(oh3, vc)
        return carry

    lax.fori_loop(0, NCH, body3, 0)


def kernel(seq, attn_mask, Wq, bq, Wk, bk, Wv, bv, planes):
    del attn_mask
    N, L, D = seq.shape
    H = _N_HEADS
    E = D // H
    C = _N_CLUSTERS
    NH = N * H

    x = seq.reshape(N * L, D)
    bcat = jnp.concatenate([bq, bk, bv])[None, :]

    ROWS = 512
    qkv = pl.pallas_call(
        _qkv_kernel,
        grid=(N * L // ROWS,),
        in_specs=[
            pl.BlockSpec((ROWS, D), lambda i: (i, 0)),
            pl.BlockSpec((D, D), lambda i: (0, 0)),
            pl.BlockSpec((D, D), lambda i: (0, 0)),
            pl.BlockSpec((D, D), lambda i: (0, 0)),
            pl.BlockSpec((1, 3 * D), lambda i: (0, 0)),
        ],
        out_specs=pl.BlockSpec((ROWS, 3 * D), lambda i: (i, 0)),
        out_shape=jax.ShapeDtypeStruct((N * L, 3 * D), jnp.float32),
    )(x, Wq, Wk, Wv, bcat)

    qkv4 = qkv.reshape(N * L, 3 * H, 1, E)

    pt = planes[:, :E].T
    init_idx = jnp.linspace(0, L - 1, C).astype(jnp.int32)
    ohinit = (init_idx[:, None] == jnp.arange(L)[None, :]).astype(jnp.float32)

    out = pl.pallas_call(
        _cluster_attn_kernel,
        grid=(NH,),
        in_specs=[
            pl.BlockSpec((L, 1, 1, E), lambda i: (i // H, i % H, 0, 0)),
            pl.BlockSpec((L, 1, 1, E), lambda i: (i // H, H + i % H, 0, 0)),
            pl.BlockSpec((L, 1, 1, E), lambda i: (i // H, 2 * H + i % H, 0, 0)),
            pl.BlockSpec((E, _BITS), lambda i: (0, 0)),
            pl.BlockSpec((C, L), lambda i: (0, 0)),
        ],
        out_specs=pl.BlockSpec((1, L, E), lambda i: (i, 0, 0)),
        out_shape=jax.ShapeDtypeStruct((NH, L, E), jnp.float32),
    )(qkv4, qkv4, qkv4, pt, ohinit)

    return out.reshape(N, H, L, E)


# speedup vs baseline: 1.2168x; 1.2168x over previous
import math

import jax
import jax.numpy as jnp
from jax import lax
from jax.experimental import pallas as pl

_N_HEADS = 16
_D_MODEL = 1024
_N_CLUSTERS = 256
_BITS = 32


def _qkv_kernel(x_ref, wq_ref, wk_ref, wv_ref, b_ref, o_ref):
    x = x_ref[...]
    D = x.shape[1]
    o_ref[:, 0:D] = _dot_t(x, wq_ref[...]) + b_ref[:, 0:D]
    o_ref[:, D:2 * D] = _dot_t(x, wk_ref[...]) + b_ref[:, D:2 * D]
    o_ref[:, 2 * D:3 * D] = _dot_t(x, wv_ref[...]) + b_ref[:, 2 * D:3 * D]


def _dot(a, b):
    return lax.dot_general(a, b, (((1,), (0,)), ((), ())),
                           preferred_element_type=jnp.float32)


def _dot_t(a, b):
    return lax.dot_general(a, b, (((1,), (1,)), ((), ())),
                           preferred_element_type=jnp.float32)


def _cluster_attn_kernel(q_ref, k_ref, v_ref, pt_ref, ohinit_ref, o_ref):
    L = q_ref.shape[1]
    E = q_ref.shape[2]
    C = _N_CLUSTERS
    CH = 2048
    NCH = L // CH
    f32 = jnp.float32
    pt = pt_ref[...]

    bits_all = (_dot(q_ref[0], pt) > 0).astype(f32)
    cent = _dot(ohinit_ref[...], bits_all)

    iota_sc = lax.broadcasted_iota(jnp.int32, (CH, C), 1)
    iota_f = iota_sc.astype(f32)
    ones_col = jnp.ones((CH, 1), f32)

    def cs_row(centroids):
        return lax.dot_general(jnp.ones((1, _BITS), f32), centroids,
                               (((1,), (1,)), ((), ())),
                               preferred_element_type=f32)

    def onehot_chunk(i, centroids, csr):
        qc = q_ref[0, pl.ds(i * CH, CH), :]
        b = (_dot(qc, pt) > 0).astype(f32)
        xc = _dot_t(b, centroids)
        dd = (csr - 2.0 * xc) * 256.0 + iota_f
        mn = jnp.min(dd, axis=1, keepdims=True)
        return (dd == mn).astype(f32), b, qc

    csr1 = cs_row(cent)

    def body1(i, acc):
        oh, b, _ = onehot_chunk(i, cent, csr1)
        rhs = jnp.concatenate([b, ones_col], axis=1)
        return acc + lax.dot_general(oh, rhs, (((0,), (0,)), ((), ())),
                                     preferred_element_type=f32)

    acc1 = lax.fori_loop(0, NCH, body1, jnp.zeros((C, _BITS + 1), f32))
    sums1 = acc1[:, :_BITS]
    counts1 = acc1[:, _BITS:]
    cent2 = jnp.where(counts1 > 0, (2.0 * sums1 > counts1).astype(f32), cent)

    csr2 = cs_row(cent2)

    def body2(i, carry):
        acc, cnt_row = carry
        oh, _, qc = onehot_chunk(i, cent2, csr2)
        rhs = jnp.concatenate([qc, ones_col], axis=1)
        acc = acc + lax.dot_general(oh, rhs, (((0,), (0,)), ((), ())),
                                    preferred_element_type=f32)
        return acc, cnt_row + jnp.sum(oh, axis=0, keepdims=True)

    acc2, cnt_row = lax.fori_loop(
        0, NCH, body2,
        (jnp.zeros((C, E + 1), f32), jnp.zeros((1, C), f32)))
    qgsum = acc2[:, :E]
    counts2 = acc2[:, E:]
    factors = jnp.where(counts2 > 0, 1.0 / jnp.maximum(counts2, 1.0), 0.0)
    qg = qgsum * factors

    logits = _dot_t(qg, k_ref[0]) * (1.0 / math.sqrt(E))
    m = jnp.max(logits, axis=1, keepdims=True)
    e = jnp.exp(logits - m)
    attn = e / jnp.sum(e, axis=1, keepdims=True)
    vc = _dot(attn, v_ref[0])

    tri = (lax.broadcasted_iota(jnp.int32, (C, C), 0)
           <= lax.broadcasted_iota(jnp.int32, (C, C), 1)).astype(f32)
    cum = _dot(cnt_row, tri).astype(jnp.int32)

    def body3(i, carry):
        base = i * CH
        li = lax.broadcasted_iota(jnp.int32, (CH, C), 0) + base
        sc = jnp.sum((cum <= li).astype(jnp.int32), axis=1)
        oh3 = (sc[:, None] == iota_sc).astype(f32)
        o_ref[0, pl.ds(base, CH), :] = _dot(oh3, vc)
        return carry

    lax.fori_loop(0, NCH, body3, 0)


def kernel(seq, attn_mask, Wq, bq, Wk, bk, Wv, bv, planes):
    del attn_mask
    N, L, D = seq.shape
    H = _N_HEADS
    E = D // H
    C = _N_CLUSTERS
    NH = N * H

    x = seq.reshape(N * L, D)
    bcat = jnp.concatenate([bq, bk, bv])[None, :]

    ROWS = 512
    qkv = pl.pallas_call(
        _qkv_kernel,
        grid=(N * L // ROWS,),
        in_specs=[
            pl.BlockSpec((ROWS, D), lambda i: (i, 0)),
            pl.BlockSpec((D, D), lambda i: (0, 0)),
            pl.BlockSpec((D, D), lambda i: (0, 0)),
            pl.BlockSpec((D, D), lambda i: (0, 0)),
            pl.BlockSpec((1, 3 * D), lambda i: (0, 0)),
        ],
        out_specs=pl.BlockSpec((ROWS, 3 * D), lambda i: (i, 0)),
        out_shape=jax.ShapeDtypeStruct((N * L, 3 * D), jnp.float32),
    )(x, Wq, Wk, Wv, bcat)

    def heads(a):
        return a.reshape(N, L, H, E).transpose(0, 2, 1, 3).reshape(NH, L, E)

    Q = heads(qkv[:, :D].reshape(N, L, D))
    K = heads(qkv[:, D:2 * D].reshape(N, L, D))
    V = heads(qkv[:, 2 * D:].reshape(N, L, D))

    pt = planes[:, :E].T
    init_idx = jnp.linspace(0, L - 1, C).astype(jnp.int32)
    ohinit = (init_idx[:, None] == jnp.arange(L)[None, :]).astype(jnp.float32)

    out = pl.pallas_call(
        _cluster_attn_kernel,
        grid=(NH,),
        in_specs=[
            pl.BlockSpec((1, L, E), lambda i: (i, 0, 0)),
            pl.BlockSpec((1, L, E), lambda i: (i, 0, 0)),
            pl.BlockSpec((1, L, E), lambda i: (i, 0, 0)),
            pl.BlockSpec((E, _BITS), lambda i: (0, 0)),
            pl.BlockSpec((C, L), lambda i: (0, 0)),
        ],
        out_specs=pl.BlockSpec((1, L, E), lambda i: (i, 0, 0)),
        out_shape=jax.ShapeDtypeStruct((NH, L, E), jnp.float32),
    )(Q, K, V, pt, ohinit)

    return out.reshape(N, H, L, E)
